# Initial kernel scaffold; baseline (speedup 1.0000x reference)
import jax
import jax.numpy as jnp
from jax.experimental import pallas as pl


def kernel(x1, edge_index1, edge_attr1, batch1, x2, edge_index2, edge_attr2, batch2, We, W1, b1, Wc, bc):
    raise NotImplementedError("write your pallas kernel here")


# speedup vs baseline: 1.6750x; 1.6750x over previous
import functools

import jax
import jax.numpy as jnp
from jax import lax
from jax.experimental import pallas as pl
from jax.experimental.pallas import tpu as pltpu
from jax.experimental.pallas import tpu_sc as plsc

_N = 10000
_E = 320000
_D = 128
_DE = 16
_G = 64

_TILES = 16
_EPT = _E // _TILES
_K = 80
_NCHUNK = _EPT // _K
_RPT = 624


def _sc_body(x_h, src_h, dst_h, m_h, zx_h, seq_h,
             aggx_o,
             src_v, dst_v, rows_v, mrow_v, agg_sh, sem):
    c = lax.axis_index("c")
    s = lax.axis_index("s")
    r0 = s * _RPT

    def fill_seq(base):
        pltpu.sync_copy(seq_h.at[pl.ds(base, _K)], src_v)

    pltpu.sync_copy(zx_h, rows_v)
    for j in range(8):
        fill_seq(r0 + j * _K)
        pltpu.sync_copy(rows_v, agg_sh.at[src_v])

    plsc.subcore_barrier()

    tbase = c * _E + s * _EPT

    def chunk(i, carry):
        base = tbase + i * _K
        pltpu.sync_copy(src_h.at[pl.ds(base, _K)], src_v)
        pltpu.sync_copy(dst_h.at[pl.ds(base, _K)], dst_v)
        pltpu.sync_copy(m_h.at[pl.ds(base, _K)], mrow_v)
        pltpu.async_copy(x_h.at[src_v], rows_v, sem).wait()
        pltpu.sync_copy(rows_v, agg_sh.at[dst_v], add=True)
        pltpu.sync_copy(mrow_v, agg_sh.at[dst_v], add=True)
        return carry

    lax.fori_loop(0, _NCHUNK, chunk, 0)

    plsc.subcore_barrier()

    for j in range(8):
        rj = r0 + j * _K
        fill_seq(rj)
        pltpu.sync_copy(agg_sh.at[src_v], rows_v)
        pltpu.sync_copy(rows_v, aggx_o.at[c, pl.ds(rj, _K)])


_sc_agg = functools.partial(
    pl.kernel,
    mesh=plsc.VectorSubcoreMesh(core_axis_name="c", subcore_axis_name="s"),
    out_type=jax.ShapeDtypeStruct((2, _N, _D), jnp.float32),
    scratch_types=[
        pltpu.VMEM((_K,), jnp.int32),
        pltpu.VMEM((_K,), jnp.int32),
        pltpu.VMEM((_K, _D), jnp.float32),
        pltpu.VMEM((_K, _D), jnp.float32),
        pltpu.VMEM_SHARED((_N, _D), jnp.float32),
        pltpu.SemaphoreType.DMA,
    ],
)(_sc_body)


_BM = 4000


def _bf(v):
    return v.astype(jnp.bfloat16).astype(jnp.float32)


def _mw_body(ea_ref, We_ref, m_ref):
    ea = _bf(ea_ref[...])
    We = _bf(We_ref[...])
    acc = ea[:, 0:1] * We[0:1, :]
    for k in range(1, _DE):
        acc = acc + ea[:, k:k + 1] * We[k:k + 1, :]
    m_ref[...] = acc


_mw = pl.pallas_call(
    _mw_body,
    grid=(2 * _E // _BM,),
    in_specs=[
        pl.BlockSpec((_BM, _DE), lambda i: (i, 0)),
        pl.BlockSpec((_DE, _D), lambda i: (0, 0)),
    ],
    out_specs=pl.BlockSpec((_BM, _D), lambda i: (i, 0)),
    out_shape=jax.ShapeDtypeStruct((2 * _E, _D), jnp.float32),
)


_BN = 2000
_NSTEP = _N // _BN


def _tc_body(x1_ref, x2_ref, aggx_ref, bt1_ref, bt2_ref,
             W1_ref, b1_ref, Wc_ref, bc_ref, out_ref,
             p1_acc, p2_acc, c1_acc, c2_acc):
    i = pl.program_id(0)

    @pl.when(i == 0)
    def _init():
        p1_acc[...] = jnp.zeros_like(p1_acc)
        p2_acc[...] = jnp.zeros_like(p2_acc)
        c1_acc[...] = jnp.zeros_like(c1_acc)
        c2_acc[...] = jnp.zeros_like(c2_acc)

    W1 = _bf(W1_ref[...])
    b1 = b1_ref[...]
    gids = lax.broadcasted_iota(jnp.int32, (1, _G), 1)
    ones_bn = jnp.ones((_BN, 1), jnp.float32)
    tdot = (((0,), (0,)), ((), ()))
    hi = lax.Precision.HIGHEST

    def accum(x, agg, bt, p_acc, c_acc):
        t = _bf(x + agg)
        h = jnp.maximum(jnp.dot(t, W1, preferred_element_type=jnp.float32,
                                precision=hi) + b1, 0.0)
        onehot = (bt == gids).astype(jnp.float32)
        p_acc[...] += lax.dot_general(onehot, h, tdot,
                                      preferred_element_type=jnp.float32,
                                      precision=hi)
        c_acc[...] += lax.dot_general(onehot, ones_bn, tdot,
                                      preferred_element_type=jnp.float32,
                                      precision=hi)

    accum(x1_ref[...], aggx_ref[0], bt1_ref[...], p1_acc, c1_acc)
    accum(x2_ref[...], aggx_ref[1], bt2_ref[...], p2_acc, c2_acc)

    @pl.when(i == _NSTEP - 1)
    def _final():
        p1 = p1_acc[...] / jnp.maximum(c1_acc[...], 1.0)
        p2 = p2_acc[...] / jnp.maximum(c2_acc[...], 1.0)
        ge = _bf(jnp.concatenate([p1, p2], axis=1))
        out_ref[...] = (
            jnp.sum(ge * _bf(Wc_ref[...]), axis=1, keepdims=True) + bc_ref[...]
        )


_tc_dense = pl.pallas_call(
    _tc_body,
    grid=(_NSTEP,),
    in_specs=[
        pl.BlockSpec((_BN, _D), lambda i: (i, 0)),
        pl.BlockSpec((_BN, _D), lambda i: (i, 0)),
        pl.BlockSpec((2, _BN, _D), lambda i: (0, i, 0)),
        pl.BlockSpec((_BN, 1), lambda i: (i, 0)),
        pl.BlockSpec((_BN, 1), lambda i: (i, 0)),
        pl.BlockSpec((_D, _D), lambda i: (0, 0)),
        pl.BlockSpec((1, _D), lambda i: (0, 0)),
        pl.BlockSpec((1, 2 * _D), lambda i: (0, 0)),
        pl.BlockSpec((1, 1), lambda i: (0, 0)),
    ],
    out_specs=pl.BlockSpec((_G, 1), lambda i: (0, 0)),
    out_shape=jax.ShapeDtypeStruct((_G, 1), jnp.float32),
    scratch_shapes=[
        pltpu.VMEM((_G, _D), jnp.float32),
        pltpu.VMEM((_G, _D), jnp.float32),
        pltpu.VMEM((_G, 1), jnp.float32),
        pltpu.VMEM((_G, 1), jnp.float32),
    ],
)


def kernel(x1, edge_index1, edge_attr1, batch1, x2, edge_index2, edge_attr2,
           batch2, We, W1, b1, Wc, bc):
    zx = jnp.zeros((_K, _D), jnp.float32)
    x_all = jnp.concatenate([x1, x2], axis=0)
    src_all = jnp.concatenate([edge_index1[0], edge_index2[0] + _N])
    dst_all = jnp.concatenate([edge_index1[1], edge_index2[1]])
    ea_all = jnp.concatenate([edge_attr1, edge_attr2], axis=0)
    seq = jnp.arange(_N + 240, dtype=jnp.int32)
    m_all = _mw(ea_all, We)
    aggx = _sc_agg(x_all, src_all, dst_all, m_all, zx, seq)
    out = _tc_dense(x1, x2, aggx,
                    batch1.reshape(_N, 1), batch2.reshape(_N, 1),
                    W1, b1.reshape(1, _D), Wc.reshape(1, 2 * _D),
                    bc.reshape(1, 1))
    return out


# speedup vs baseline: 2.0775x; 1.2403x over previous
import functools

import jax
import jax.numpy as jnp
from jax import lax
from jax.experimental import pallas as pl
from jax.experimental.pallas import tpu as pltpu
from jax.experimental.pallas import tpu_sc as plsc

_N = 10000
_E = 320000
_D = 128
_DE = 16
_G = 64

_TILES = 16
_EPT = _E // _TILES
_K = 80
_NCHUNK = _EPT // _K
_RPT = 624


def _sc_body(x_h, src_h, dst_h, m_h, zx_h, seq_h,
             aggx_o,
             src_v, dst_v, rows_v, mrow_v,
             src2_v, dst2_v, mrow2_v, agg_sh, sem, psem_a, psem_b):
    c = lax.axis_index("c")
    s = lax.axis_index("s")
    r0 = s * _RPT

    def fill_seq(base):
        pltpu.sync_copy(seq_h.at[pl.ds(base, _K)], src_v)

    pltpu.sync_copy(zx_h, rows_v)
    for j in range(8):
        fill_seq(r0 + j * _K)
        pltpu.sync_copy(rows_v, agg_sh.at[src_v])

    plsc.subcore_barrier()

    tbase = c * _E + s * _EPT

    def prefetch(i, sv, dv, mv, psem):
        base = tbase + i * _K
        pltpu.async_copy(src_h.at[pl.ds(base, _K)], sv, psem)
        pltpu.async_copy(dst_h.at[pl.ds(base, _K)], dv, psem)
        pltpu.async_copy(m_h.at[pl.ds(base, _K)], mv, psem)

    def wait_pf(sv, dv, mv, psem):
        pltpu.make_async_copy(src_h.at[pl.ds(0, _K)], sv, psem).wait()
        pltpu.make_async_copy(dst_h.at[pl.ds(0, _K)], dv, psem).wait()
        pltpu.make_async_copy(m_h.at[pl.ds(0, _K)], mv, psem).wait()

    def process(sv, dv, mv):
        pltpu.async_copy(x_h.at[sv], rows_v, sem).wait()
        pltpu.sync_copy(rows_v, agg_sh.at[dv], add=True)
        pltpu.sync_copy(mv, agg_sh.at[dv], add=True)

    prefetch(0, src_v, dst_v, mrow_v, psem_a)

    def pair(i, carry):
        i1 = 2 * i + 1
        i2 = jnp.minimum(2 * i + 2, _NCHUNK - 1)
        wait_pf(src_v, dst_v, mrow_v, psem_a)
        prefetch(i1, src2_v, dst2_v, mrow2_v, psem_b)
        process(src_v, dst_v, mrow_v)
        wait_pf(src2_v, dst2_v, mrow2_v, psem_b)
        prefetch(i2, src_v, dst_v, mrow_v, psem_a)
        process(src2_v, dst2_v, mrow2_v)
        return carry

    lax.fori_loop(0, _NCHUNK // 2, pair, 0)
    wait_pf(src_v, dst_v, mrow_v, psem_a)

    plsc.subcore_barrier()

    for j in range(8):
        rj = r0 + j * _K
        fill_seq(rj)
        pltpu.sync_copy(agg_sh.at[src_v], rows_v)
        pltpu.sync_copy(rows_v, aggx_o.at[c, pl.ds(rj, _K)])


_sc_agg = functools.partial(
    pl.kernel,
    mesh=plsc.VectorSubcoreMesh(core_axis_name="c", subcore_axis_name="s"),
    out_type=jax.ShapeDtypeStruct((2, _N, _D), jnp.float32),
    scratch_types=[
        pltpu.VMEM((_K,), jnp.int32),
        pltpu.VMEM((_K,), jnp.int32),
        pltpu.VMEM((_K, _D), jnp.float32),
        pltpu.VMEM((_K, _D), jnp.float32),
        pltpu.VMEM((_K,), jnp.int32),
        pltpu.VMEM((_K,), jnp.int32),
        pltpu.VMEM((_K, _D), jnp.float32),
        pltpu.VMEM_SHARED((_N, _D), jnp.float32),
        pltpu.SemaphoreType.DMA,
        pltpu.SemaphoreType.DMA,
        pltpu.SemaphoreType.DMA,
    ],
)(_sc_body)


_BM = 4000


def _bf(v):
    return v.astype(jnp.bfloat16).astype(jnp.float32)


def _mw_body(ea_ref, We_ref, m_ref):
    ea = _bf(ea_ref[...])
    We = _bf(We_ref[...])
    acc = ea[:, 0:1] * We[0:1, :]
    for k in range(1, _DE):
        acc = acc + ea[:, k:k + 1] * We[k:k + 1, :]
    m_ref[...] = acc


_mw = pl.pallas_call(
    _mw_body,
    grid=(2 * _E // _BM,),
    in_specs=[
        pl.BlockSpec((_BM, _DE), lambda i: (i, 0)),
        pl.BlockSpec((_DE, _D), lambda i: (0, 0)),
    ],
    out_specs=pl.BlockSpec((_BM, _D), lambda i: (i, 0)),
    out_shape=jax.ShapeDtypeStruct((2 * _E, _D), jnp.float32),
)


_BN = 2000
_NSTEP = _N // _BN


def _tc_body(x1_ref, x2_ref, aggx_ref, bt1_ref, bt2_ref,
             W1_ref, b1_ref, Wc_ref, bc_ref, out_ref,
             p1_acc, p2_acc, c1_acc, c2_acc):
    i = pl.program_id(0)

    @pl.when(i == 0)
    def _init():
        p1_acc[...] = jnp.zeros_like(p1_acc)
        p2_acc[...] = jnp.zeros_like(p2_acc)
        c1_acc[...] = jnp.zeros_like(c1_acc)
        c2_acc[...] = jnp.zeros_like(c2_acc)

    W1 = _bf(W1_ref[...])
    b1 = b1_ref[...]
    gids = lax.broadcasted_iota(jnp.int32, (1, _G), 1)
    ones_bn = jnp.ones((_BN, 1), jnp.float32)
    tdot = (((0,), (0,)), ((), ()))
    hi = lax.Precision.HIGHEST

    def accum(x, agg, bt, p_acc, c_acc):
        t = _bf(x + agg)
        h = jnp.maximum(jnp.dot(t, W1, preferred_element_type=jnp.float32,
                                precision=hi) + b1, 0.0)
        onehot = (bt == gids).astype(jnp.float32)
        p_acc[...] += lax.dot_general(onehot, h, tdot,
                                      preferred_element_type=jnp.float32,
                                      precision=hi)
        c_acc[...] += lax.dot_general(onehot, ones_bn, tdot,
                                      preferred_element_type=jnp.float32,
                                      precision=hi)

    accum(x1_ref[...], aggx_ref[0], bt1_ref[...], p1_acc, c1_acc)
    accum(x2_ref[...], aggx_ref[1], bt2_ref[...], p2_acc, c2_acc)

    @pl.when(i == _NSTEP - 1)
    def _final():
        p1 = p1_acc[...] / jnp.maximum(c1_acc[...], 1.0)
        p2 = p2_acc[...] / jnp.maximum(c2_acc[...], 1.0)
        ge = _bf(jnp.concatenate([p1, p2], axis=1))
        out_ref[...] = (
            jnp.sum(ge * _bf(Wc_ref[...]), axis=1, keepdims=True) + bc_ref[...]
        )


_tc_dense = pl.pallas_call(
    _tc_body,
    grid=(_NSTEP,),
    in_specs=[
        pl.BlockSpec((_BN, _D), lambda i: (i, 0)),
        pl.BlockSpec((_BN, _D), lambda i: (i, 0)),
        pl.BlockSpec((2, _BN, _D), lambda i: (0, i, 0)),
        pl.BlockSpec((_BN, 1), lambda i: (i, 0)),
        pl.BlockSpec((_BN, 1), lambda i: (i, 0)),
        pl.BlockSpec((_D, _D), lambda i: (0, 0)),
        pl.BlockSpec((1, _D), lambda i: (0, 0)),
        pl.BlockSpec((1, 2 * _D), lambda i: (0, 0)),
        pl.BlockSpec((1, 1), lambda i: (0, 0)),
    ],
    out_specs=pl.BlockSpec((_G, 1), lambda i: (0, 0)),
    out_shape=jax.ShapeDtypeStruct((_G, 1), jnp.float32),
    scratch_shapes=[
        pltpu.VMEM((_G, _D), jnp.float32),
        pltpu.VMEM((_G, _D), jnp.float32),
        pltpu.VMEM((_G, 1), jnp.float32),
        pltpu.VMEM((_G, 1), jnp.float32),
    ],
)


def kernel(x1, edge_index1, edge_attr1, batch1, x2, edge_index2, edge_attr2,
           batch2, We, W1, b1, Wc, bc):
    zx = jnp.zeros((_K, _D), jnp.float32)
    x_all = jnp.concatenate([x1, x2], axis=0)
    src_all = jnp.concatenate([edge_index1[0], edge_index2[0] + _N])
    dst_all = jnp.concatenate([edge_index1[1], edge_index2[1]])
    ea_all = jnp.concatenate([edge_attr1, edge_attr2], axis=0)
    seq = jnp.arange(_N + 240, dtype=jnp.int32)
    m_all = _mw(ea_all, We)
    aggx = _sc_agg(x_all, src_all, dst_all, m_all, zx, seq)
    out = _tc_dense(x1, x2, aggx,
                    batch1.reshape(_N, 1), batch2.reshape(_N, 1),
                    W1, b1.reshape(1, _D), Wc.reshape(1, 2 * _D),
                    bc.reshape(1, 1))
    return out


# speedup vs baseline: 2.0967x; 1.0092x over previous
import functools

import jax
import jax.numpy as jnp
from jax import lax
from jax.experimental import pallas as pl
from jax.experimental.pallas import tpu as pltpu
from jax.experimental.pallas import tpu_sc as plsc

_N = 10000
_E = 320000
_D = 128
_DE = 16
_G = 64

_TILES = 16
_EPT = _E // _TILES
_K = 80
_NCHUNK = _EPT // _K
_RPT = 624


def _sc_body(x_h, src_h, dst_h, m_h, zx_h, seq_h,
             aggx_o,
             src_v, dst_v, rows_v, mrow_v,
             src2_v, dst2_v, mrow2_v, agg_sh, sem, psem_a, psem_b):
    c = lax.axis_index("c")
    s = lax.axis_index("s")
    r0 = s * _RPT

    def fill_seq(base):
        pltpu.sync_copy(seq_h.at[pl.ds(base, _K)], src_v)

    pltpu.sync_copy(zx_h, rows_v)
    for j in range(8):
        fill_seq(r0 + j * _K)
        pltpu.sync_copy(rows_v, agg_sh.at[src_v])

    plsc.subcore_barrier()

    tbase = c * _E + s * _EPT

    def prefetch(i, sv, dv, mv, psem):
        base = tbase + i * _K
        pltpu.async_copy(src_h.at[pl.ds(base, _K)], sv, psem)
        pltpu.async_copy(dst_h.at[pl.ds(base, _K)], dv, psem)
        pltpu.async_copy(m_h.at[pl.ds(base, _K)], mv, psem)

    def wait_pf(sv, dv, mv, psem):
        pltpu.make_async_copy(src_h.at[pl.ds(0, _K)], sv, psem).wait()
        pltpu.make_async_copy(dst_h.at[pl.ds(0, _K)], dv, psem).wait()
        pltpu.make_async_copy(m_h.at[pl.ds(0, _K)], mv, psem).wait()

    def process(sv, dv, mv):
        pltpu.async_copy(x_h.at[sv], rows_v, sem).wait()
        h1 = pltpu.async_copy(rows_v, agg_sh.at[dv], sem, add=True)
        h2 = pltpu.async_copy(mv, agg_sh.at[dv], sem, add=True)
        h1.wait()
        h2.wait()

    prefetch(0, src_v, dst_v, mrow_v, psem_a)

    def pair(i, carry):
        i1 = 2 * i + 1
        i2 = jnp.minimum(2 * i + 2, _NCHUNK - 1)
        wait_pf(src_v, dst_v, mrow_v, psem_a)
        prefetch(i1, src2_v, dst2_v, mrow2_v, psem_b)
        process(src_v, dst_v, mrow_v)
        wait_pf(src2_v, dst2_v, mrow2_v, psem_b)
        prefetch(i2, src_v, dst_v, mrow_v, psem_a)
        process(src2_v, dst2_v, mrow2_v)
        return carry

    lax.fori_loop(0, _NCHUNK // 2, pair, 0)
    wait_pf(src_v, dst_v, mrow_v, psem_a)

    plsc.subcore_barrier()

    for j in range(8):
        rj = r0 + j * _K
        fill_seq(rj)
        pltpu.sync_copy(agg_sh.at[src_v], rows_v)
        pltpu.sync_copy(rows_v, aggx_o.at[c, pl.ds(rj, _K)])


_sc_agg = functools.partial(
    pl.kernel,
    mesh=plsc.VectorSubcoreMesh(core_axis_name="c", subcore_axis_name="s"),
    out_type=jax.ShapeDtypeStruct((2, _N, _D), jnp.float32),
    scratch_types=[
        pltpu.VMEM((_K,), jnp.int32),
        pltpu.VMEM((_K,), jnp.int32),
        pltpu.VMEM((_K, _D), jnp.float32),
        pltpu.VMEM((_K, _D), jnp.float32),
        pltpu.VMEM((_K,), jnp.int32),
        pltpu.VMEM((_K,), jnp.int32),
        pltpu.VMEM((_K, _D), jnp.float32),
        pltpu.VMEM_SHARED((_N, _D), jnp.float32),
        pltpu.SemaphoreType.DMA,
        pltpu.SemaphoreType.DMA,
        pltpu.SemaphoreType.DMA,
    ],
)(_sc_body)


_BM = 4000


def _bf(v):
    return v.astype(jnp.bfloat16).astype(jnp.float32)


def _mw_body(ea_ref, We_ref, m_ref):
    ea = _bf(ea_ref[...])
    We = _bf(We_ref[...])
    acc = ea[:, 0:1] * We[0:1, :]
    for k in range(1, _DE):
        acc = acc + ea[:, k:k + 1] * We[k:k + 1, :]
    m_ref[...] = acc


_mw = pl.pallas_call(
    _mw_body,
    grid=(2 * _E // _BM,),
    in_specs=[
        pl.BlockSpec((_BM, _DE), lambda i: (i, 0)),
        pl.BlockSpec((_DE, _D), lambda i: (0, 0)),
    ],
    out_specs=pl.BlockSpec((_BM, _D), lambda i: (i, 0)),
    out_shape=jax.ShapeDtypeStruct((2 * _E, _D), jnp.float32),
)


_BN = 2000
_NSTEP = _N // _BN


def _tc_body(x1_ref, x2_ref, aggx_ref, bt1_ref, bt2_ref,
             W1_ref, b1_ref, Wc_ref, bc_ref, out_ref,
             p1_acc, p2_acc, c1_acc, c2_acc):
    i = pl.program_id(0)

    @pl.when(i == 0)
    def _init():
        p1_acc[...] = jnp.zeros_like(p1_acc)
        p2_acc[...] = jnp.zeros_like(p2_acc)
        c1_acc[...] = jnp.zeros_like(c1_acc)
        c2_acc[...] = jnp.zeros_like(c2_acc)

    W1 = _bf(W1_ref[...])
    b1 = b1_ref[...]
    gids = lax.broadcasted_iota(jnp.int32, (1, _G), 1)
    ones_bn = jnp.ones((_BN, 1), jnp.float32)
    tdot = (((0,), (0,)), ((), ()))
    hi = lax.Precision.HIGHEST

    def accum(x, agg, bt, p_acc, c_acc):
        t = _bf(x + agg)
        h = jnp.maximum(jnp.dot(t, W1, preferred_element_type=jnp.float32,
                                precision=hi) + b1, 0.0)
        onehot = (bt == gids).astype(jnp.float32)
        p_acc[...] += lax.dot_general(onehot, h, tdot,
                                      preferred_element_type=jnp.float32,
                                      precision=hi)
        c_acc[...] += lax.dot_general(onehot, ones_bn, tdot,
                                      preferred_element_type=jnp.float32,
                                      precision=hi)

    accum(x1_ref[...], aggx_ref[0], bt1_ref[...], p1_acc, c1_acc)
    accum(x2_ref[...], aggx_ref[1], bt2_ref[...], p2_acc, c2_acc)

    @pl.when(i == _NSTEP - 1)
    def _final():
        p1 = p1_acc[...] / jnp.maximum(c1_acc[...], 1.0)
        p2 = p2_acc[...] / jnp.maximum(c2_acc[...], 1.0)
        ge = _bf(jnp.concatenate([p1, p2], axis=1))
        out_ref[...] = (
            jnp.sum(ge * _bf(Wc_ref[...]), axis=1, keepdims=True) + bc_ref[...]
        )


_tc_dense = pl.pallas_call(
    _tc_body,
    grid=(_NSTEP,),
    in_specs=[
        pl.BlockSpec((_BN, _D), lambda i: (i, 0)),
        pl.BlockSpec((_BN, _D), lambda i: (i, 0)),
        pl.BlockSpec((2, _BN, _D), lambda i: (0, i, 0)),
        pl.BlockSpec((_BN, 1), lambda i: (i, 0)),
        pl.BlockSpec((_BN, 1), lambda i: (i, 0)),
        pl.BlockSpec((_D, _D), lambda i: (0, 0)),
        pl.BlockSpec((1, _D), lambda i: (0, 0)),
        pl.BlockSpec((1, 2 * _D), lambda i: (0, 0)),
        pl.BlockSpec((1, 1), lambda i: (0, 0)),
    ],
    out_specs=pl.BlockSpec((_G, 1), lambda i: (0, 0)),
    out_shape=jax.ShapeDtypeStruct((_G, 1), jnp.float32),
    scratch_shapes=[
        pltpu.VMEM((_G, _D), jnp.float32),
        pltpu.VMEM((_G, _D), jnp.float32),
        pltpu.VMEM((_G, 1), jnp.float32),
        pltpu.VMEM((_G, 1), jnp.float32),
    ],
)


def kernel(x1, edge_index1, edge_attr1, batch1, x2, edge_index2, edge_attr2,
           batch2, We, W1, b1, Wc, bc):
    zx = jnp.zeros((_K, _D), jnp.float32)
    x_all = jnp.concatenate([x1, x2], axis=0)
    src_all = jnp.concatenate([edge_index1[0], edge_index2[0] + _N])
    dst_all = jnp.concatenate([edge_index1[1], edge_index2[1]])
    ea_all = jnp.concatenate([edge_attr1, edge_attr2], axis=0)
    seq = jnp.arange(_N + 240, dtype=jnp.int32)
    m_all = _mw(ea_all, We)
    aggx = _sc_agg(x_all, src_all, dst_all, m_all, zx, seq)
    out = _tc_dense(x1, x2, aggx,
                    batch1.reshape(_N, 1), batch2.reshape(_N, 1),
                    W1, b1.reshape(1, _D), Wc.reshape(1, 2 * _D),
                    bc.reshape(1, 1))
    return out
